# baseline (speedup 1.0000x reference)
import jax
import jax.numpy as jnp
from jax import lax
from jax.experimental import pallas as pl
from jax.experimental.pallas import tpu as pltpu
from jax.experimental.pallas import tpu_sc as plsc

NUM_BOOKS = 1000000
EMBED_DIM = 32
BATCH = 16384
HIST = 50
TOTAL = BATCH * HIST

_info = plsc.get_sparse_core_info()
NC, NS = _info.num_cores, _info.num_subcores
NW = NC * NS
ROWS_PER_W = BATCH // NW
NB = 16
CHUNK = NB * HIST
N_CHUNKS = ROWS_PER_W // NB
IDX_PER_W = ROWS_PER_W * HIST


def _gather_body(idx_hbm, table_hbm, out_hbm,
                 idx_v, rows0, rows1, gs0, gs1, ss0, ss1):
    wid = lax.axis_index("s") * NC + lax.axis_index("c")
    w_row = wid * ROWS_PER_W

    pltpu.sync_copy(idx_hbm.at[pl.ds(wid * IDX_PER_W, IDX_PER_W)], idx_v)

    rows = (rows0, rows1)
    gs = (gs0, gs1)
    ss = (ss0, ss1)

    def gather(g, b):
        src = table_hbm.at[idx_v.at[pl.ds(g * CHUNK, CHUNK)]]
        return pltpu.async_copy(src, rows[b], gs[b])

    def store(g, b):
        dst = out_hbm.at[pl.ds(wid * IDX_PER_W + g * CHUNK, CHUNK)]
        return [pltpu.async_copy(rows[b], dst, ss[b])]

    pend_g = [None, None]
    pend_s = [[], []]
    pend_g[0] = gather(0, 0)
    for g in range(1, N_CHUNKS):
        b = g & 1
        for d in pend_s[b]:
            d.wait()
        pend_s[b] = []
        pend_g[b] = gather(g, b)
        pend_g[1 - b].wait()
        pend_s[1 - b] = store(g - 1, 1 - b)
    bl = (N_CHUNKS - 1) & 1
    pend_g[bl].wait()
    pend_s[bl] = store(N_CHUNKS - 1, bl)
    for b in (0, 1):
        for d in pend_s[b]:
            d.wait()


@jax.jit
def kernel(book_idx, embedding_books):
    flat_idx = book_idx.reshape(TOTAL).astype(jnp.int32)
    mesh = plsc.VectorSubcoreMesh(core_axis_name="c", subcore_axis_name="s")
    out = pl.kernel(
        _gather_body,
        out_type=jax.ShapeDtypeStruct((TOTAL, EMBED_DIM), jnp.float32),
        mesh=mesh,
        scratch_types=[
            pltpu.VMEM((IDX_PER_W,), jnp.int32),
            pltpu.VMEM((CHUNK, EMBED_DIM), jnp.float32),
            pltpu.VMEM((CHUNK, EMBED_DIM), jnp.float32),
            pltpu.SemaphoreType.DMA,
            pltpu.SemaphoreType.DMA,
            pltpu.SemaphoreType.DMA,
            pltpu.SemaphoreType.DMA,
        ],
        compiler_params=pltpu.CompilerParams(use_tc_tiling_on_sc=False),
    )(flat_idx, embedding_books)
    return out.reshape(BATCH, HIST, EMBED_DIM)


# speedup vs baseline: 1.6210x; 1.6210x over previous
import jax
import jax.numpy as jnp
from jax import lax
from jax.experimental import pallas as pl
from jax.experimental.pallas import tpu as pltpu
from jax.experimental.pallas import tpu_sc as plsc

NUM_BOOKS = 1000000
EMBED_DIM = 32
BATCH = 16384
HIST = 50
TOTAL = BATCH * HIST

_info = plsc.get_sparse_core_info()
NC, NS = _info.num_cores, _info.num_subcores
NW = NC * NS
ROWS_PER_W = BATCH // NW
NB = 8
CHUNK = NB * HIST
N_CHUNKS = ROWS_PER_W // NB
IDX_PER_W = ROWS_PER_W * HIST
DEPTH = 4


def _gather_body(idx_hbm, table_hbm, out_hbm, idx_v, *bufs):
    rows = bufs[:DEPTH]
    gs = bufs[DEPTH:2 * DEPTH]
    ss = bufs[2 * DEPTH:]

    wid = lax.axis_index("s") * NC + lax.axis_index("c")
    w_row = wid * ROWS_PER_W

    pltpu.sync_copy(idx_hbm.at[pl.ds(wid * IDX_PER_W, IDX_PER_W)], idx_v)

    def gather(g, b):
        src = table_hbm.at[idx_v.at[pl.ds(g * CHUNK, CHUNK)]]
        return pltpu.async_copy(src, rows[b], gs[b])

    def store(g, b):
        descs = []
        for i in range(NB):
            src = rows[b].at[pl.ds(i * HIST, HIST)]
            dst = out_hbm.at[w_row + g * NB + i]
            descs.append(pltpu.async_copy(src, dst, ss[b]))
        return descs

    pend_g = [None] * DEPTH
    pend_s = [[] for _ in range(DEPTH)]
    for g in range(N_CHUNKS):
        b = g % DEPTH
        for d in pend_s[b]:
            d.wait()
        pend_s[b] = []
        pend_g[b] = gather(g, b)
        if g >= DEPTH - 1:
            go = g - DEPTH + 1
            bo = go % DEPTH
            pend_g[bo].wait()
            pend_s[bo] = store(go, bo)
    for go in range(max(0, N_CHUNKS - DEPTH + 1), N_CHUNKS):
        bo = go % DEPTH
        pend_g[bo].wait()
        pend_s[bo] = store(go, bo)
    for b in range(DEPTH):
        for d in pend_s[b]:
            d.wait()


@jax.jit
def kernel(book_idx, embedding_books):
    flat_idx = book_idx.reshape(TOTAL).astype(jnp.int32)
    mesh = plsc.VectorSubcoreMesh(core_axis_name="c", subcore_axis_name="s")
    out = pl.kernel(
        _gather_body,
        out_type=jax.ShapeDtypeStruct((BATCH, HIST, EMBED_DIM), jnp.float32),
        mesh=mesh,
        scratch_types=(
            [pltpu.VMEM((IDX_PER_W,), jnp.int32)]
            + [pltpu.VMEM((CHUNK, EMBED_DIM), jnp.float32)] * DEPTH
            + [pltpu.SemaphoreType.DMA] * (2 * DEPTH)
        ),
        compiler_params=pltpu.CompilerParams(use_tc_tiling_on_sc=False),
    )(flat_idx, embedding_books)
    return out
